# baseline (speedup 1.0000x reference)
import functools

import jax
import jax.numpy as jnp
from jax import lax
from jax.experimental import pallas as pl
from jax.experimental.pallas import tpu as pltpu
from jax.experimental.pallas import tpu_sc as plsc

_NC = 2
_NS = 16
_NW = _NC * _NS
_D = 128
_CHUNK = 512


@functools.partial(jax.jit, static_argnames=("n",))
def _sc_lookup(idx_flat, table, n):
    per_w = n // _NW
    iters = per_w // _CHUNK
    mesh = plsc.VectorSubcoreMesh(core_axis_name="c", subcore_axis_name="s")

    @functools.partial(
        pl.kernel,
        out_type=jax.ShapeDtypeStruct((n, _D), jnp.float32),
        mesh=mesh,
        scratch_types=[
            pltpu.VMEM((_CHUNK,), jnp.int32),
            pltpu.VMEM((_CHUNK, _D), jnp.float32),
            pltpu.SemaphoreType.DMA,
        ],
    )
    def k(idx_hbm, table_hbm, out_hbm, idx_v, rows_v, sem):
        wid = lax.axis_index("s") * _NC + lax.axis_index("c")
        base = wid * per_w

        @pl.loop(0, iters)
        def _(i):
            off = base + i * _CHUNK
            pltpu.sync_copy(idx_hbm.at[pl.ds(off, _CHUNK)], idx_v)
            pltpu.async_copy(table_hbm.at[idx_v], rows_v, sem).wait()
            pltpu.sync_copy(rows_v, out_hbm.at[pl.ds(off, _CHUNK)])

    return k(idx_flat, table)


def kernel(inputs, table):
    b, l = inputs.shape
    n = b * l
    table = table.at[2].set(0.0)
    out = _sc_lookup(inputs.reshape(n), table, n)
    return out.reshape(b, l, _D)


# speedup vs baseline: 29.8869x; 29.8869x over previous
import functools

import jax
import jax.numpy as jnp
from jax import lax
from jax.experimental import pallas as pl
from jax.experimental.pallas import tpu as pltpu
from jax.experimental.pallas import tpu_sc as plsc

_NC = 2
_NS = 16
_NW = _NC * _NS
_D = 128
_CHUNK = 800


@functools.partial(jax.jit, static_argnames=("n",))
def _sc_lookup(idx_flat, table, n):
    per_w = n // _NW
    iters = per_w // _CHUNK
    mesh = plsc.VectorSubcoreMesh(core_axis_name="c", subcore_axis_name="s")

    @functools.partial(
        pl.kernel,
        out_type=jax.ShapeDtypeStruct((n, _D), jnp.float32),
        mesh=mesh,
        scratch_types=[
            pltpu.VMEM((_CHUNK,), jnp.int32),
            pltpu.VMEM((_CHUNK, _D), jnp.float32),
            pltpu.VMEM_SHARED((4, _D), jnp.float32),
            pltpu.SemaphoreType.DMA,
        ],
    )
    def k(idx_hbm, table_hbm, out_hbm, idx_v, rows_v, table_sh, sem):
        sid = lax.axis_index("s")
        wid = sid * _NC + lax.axis_index("c")
        base = wid * per_w

        @pl.when(sid == 0)
        def _():
            pltpu.sync_copy(table_hbm, table_sh)

        plsc.subcore_barrier()

        @pl.loop(0, iters)
        def _(i):
            off = base + i * _CHUNK
            pltpu.sync_copy(idx_hbm.at[pl.ds(off, _CHUNK)], idx_v)
            pltpu.async_copy(table_sh.at[idx_v], rows_v, sem).wait()
            pltpu.sync_copy(rows_v, out_hbm.at[pl.ds(off, _CHUNK)])

    return k(idx_flat, table)


def kernel(inputs, table):
    b, l = inputs.shape
    n = b * l
    table = table.at[2].set(0.0)
    out = _sc_lookup(inputs.reshape(n), table, n)
    return out.reshape(b, l, _D)


# speedup vs baseline: 48.3728x; 1.6185x over previous
import functools

import jax
import jax.numpy as jnp
from jax import lax
from jax.experimental import pallas as pl
from jax.experimental.pallas import tpu as pltpu
from jax.experimental.pallas import tpu_sc as plsc

_NC = 2
_NS = 16
_NW = _NC * _NS
_D = 128
_C = 400
_SUPER = 25600
_CPS = _SUPER // _C


@functools.partial(jax.jit, static_argnames=("n",))
def _sc_lookup(idx_flat, table, n):
    per_w = n // _NW
    n_super = per_w // _SUPER
    mesh = plsc.VectorSubcoreMesh(core_axis_name="c", subcore_axis_name="s")

    @functools.partial(
        pl.kernel,
        out_type=jax.ShapeDtypeStruct((n, _D), jnp.float32),
        mesh=mesh,
        scratch_types=[
            pltpu.VMEM((_SUPER,), jnp.int32),
            pltpu.VMEM((_C, _D), jnp.float32),
            pltpu.VMEM((_C, _D), jnp.float32),
            pltpu.VMEM_SHARED((4, _D), jnp.float32),
            pltpu.SemaphoreType.DMA,
            pltpu.SemaphoreType.DMA,
            pltpu.SemaphoreType.DMA,
            pltpu.SemaphoreType.DMA,
        ],
    )
    def k(idx_hbm, table_hbm, out_hbm, idx_v, rows0, rows1,
          table_sh, g0, g1, s0, s1):
        sid = lax.axis_index("s")
        wid = sid * _NC + lax.axis_index("c")
        base = wid * per_w
        rows = (rows0, rows1)
        gsem = (g0, g1)
        ssem = (s0, s1)

        @pl.when(sid == 0)
        def _():
            pltpu.sync_copy(table_hbm, table_sh)

        plsc.subcore_barrier()

        def gstart(j, b):
            pltpu.async_copy(
                table_sh.at[idx_v.at[pl.ds(j * _C, _C)]], rows[b], gsem[b])

        def gwait(b):
            pltpu.make_async_copy(
                table_sh.at[idx_v.at[pl.ds(0, _C)]], rows[b], gsem[b]).wait()

        def sstart(off, b):
            pltpu.async_copy(rows[b], out_hbm.at[pl.ds(off, _C)], ssem[b])

        def swait(b):
            pltpu.make_async_copy(
                rows[b], out_hbm.at[pl.ds(base, _C)], ssem[b]).wait()

        @pl.loop(0, n_super)
        def _(s):
            sbase = base + s * _SUPER
            pltpu.sync_copy(idx_hbm.at[pl.ds(sbase, _SUPER)], idx_v)
            gstart(0, 0)

            @pl.loop(0, _CPS, step=2)
            def _(j):
                @pl.when(j > 0)
                def _():
                    swait(1)
                gstart(j + 1, 1)
                gwait(0)
                sstart(sbase + j * _C, 0)
                swait(0)

                @pl.when(j + 2 < _CPS)
                def _():
                    gstart(j + 2, 0)
                gwait(1)
                sstart(sbase + (j + 1) * _C, 1)

            swait(1)

    return k(idx_flat, table)


def kernel(inputs, table):
    b, l = inputs.shape
    n = b * l
    table = table.at[2].set(0.0)
    out = _sc_lookup(inputs.reshape(n), table, n)
    return out.reshape(b, l, _D)
